# Initial kernel scaffold; baseline (speedup 1.0000x reference)
import jax
import jax.numpy as jnp
from jax.experimental import pallas as pl


def kernel(x, edge_index, edge_attr, batch, emb, params):
    raise NotImplementedError("write your pallas kernel here")


# speedup vs baseline: 1.6635x; 1.6635x over previous
import functools

import jax
import jax.numpy as jnp
from jax import lax
from jax.experimental import pallas as pl
from jax.experimental.pallas import tpu as pltpu
from jax.experimental.pallas import tpu_sc as plsc

N_FEAT = 64
HALF = 32
N_RBF = 10
N_CONV = 3
N_NODES = 50000
N_EDGES = 800000
N_GRAPHS = 256

ROW_W = 128
E_PAD = 819200
N_ROWS = E_PAD // ROW_W
ROWS_PER_SUB = N_ROWS // 16
CH = 2
N_CHUNKS = ROWS_PER_SUB // CH
ACC_STRIPE = 3128
ACC_ROWS = 16 * ACC_STRIPE

E_BLK = 3200
R_BLK = 2000
P_BLK = 2000


def _softplus(v):
    return jnp.maximum(v, 0.0) + jnp.log(1.0 + jnp.exp(-jnp.abs(v)))


def _gate_body(d_ref, wep_ref, bes_ref, out_ref):
    step = 6.0 / (N_RBF - 1)
    coeff = -0.5 / step**2
    d = d_ref[...]
    lane = lax.broadcasted_iota(jnp.int32, (E_BLK, 128), 1).astype(jnp.float32)
    z = d - lane * step
    ea = jnp.exp(coeff * z * z)
    for i in range(N_CONV):
        g = lax.dot_general(ea, wep_ref[i], (((1,), (0,)), ((), ())),
                            preferred_element_type=jnp.float32,
                            precision=lax.Precision.DEFAULT)
        g = g + bes_ref[i:i + 1, :]
        out_ref[i, 0] = g[:, :HALF]
        out_ref[i, 1] = g[:, HALF:]


def _gates(d, wep, bes):
    n = E_PAD // E_BLK
    return pl.pallas_call(
        _gate_body,
        grid=(n,),
        in_specs=[
            pl.BlockSpec((E_BLK, 1), lambda j: (j, 0)),
            pl.BlockSpec((N_CONV, 128, N_FEAT), lambda j: (0, 0, 0)),
            pl.BlockSpec((N_CONV, N_FEAT), lambda j: (0, 0)),
        ],
        out_specs=pl.BlockSpec((N_CONV, 2, E_BLK, HALF),
                               lambda j: (0, 0, j, 0)),
        out_shape=jax.ShapeDtypeStruct((N_CONV, 2, E_PAD, HALF),
                                       jnp.float32),
    )(d, wep, bes)


def _embed_body(x_ref, embp_ref, wn_ref, bn_ref, h_ref, msg_ref):
    xv = x_ref[...]
    oh = (lax.broadcasted_iota(jnp.int32, (R_BLK, 128), 1) == xv
          ).astype(jnp.float32)
    h = jnp.dot(oh, embp_ref[...], preferred_element_type=jnp.float32, precision=lax.Precision.HIGHEST)
    msg = jnp.dot(h, wn_ref[...], preferred_element_type=jnp.float32,
                  precision=lax.Precision.DEFAULT) + bn_ref[...]
    h_ref[...] = h
    msg_ref[0] = msg[:, :HALF]
    msg_ref[1] = msg[:, HALF:]


def _embed(xc, embp, wn, bn):
    n = N_NODES // R_BLK
    return pl.pallas_call(
        _embed_body,
        grid=(n,),
        in_specs=[
            pl.BlockSpec((R_BLK, 1), lambda j: (j, 0)),
            pl.BlockSpec((128, N_FEAT), lambda j: (0, 0)),
            pl.BlockSpec((N_FEAT, N_FEAT), lambda j: (0, 0)),
            pl.BlockSpec((1, N_FEAT), lambda j: (0, 0)),
        ],
        out_specs=[
            pl.BlockSpec((R_BLK, N_FEAT), lambda j: (j, 0)),
            pl.BlockSpec((2, R_BLK, HALF), lambda j: (0, j, 0)),
        ],
        out_shape=[
            jax.ShapeDtypeStruct((N_NODES, N_FEAT), jnp.float32),
            jax.ShapeDtypeStruct((2, N_NODES, HALF), jnp.float32),
        ],
    )(xc, embp, wn, bn)


def _update_body(h_ref, agg_ref, wn_ref, bn_ref, h_out_ref, msg_ref):
    agg = jnp.concatenate([agg_ref[0], agg_ref[1]], axis=1)
    hn = _softplus(h_ref[...] + agg)
    msg = jnp.dot(hn, wn_ref[...], preferred_element_type=jnp.float32,
                  precision=lax.Precision.DEFAULT) + bn_ref[...]
    h_out_ref[...] = hn
    msg_ref[0] = msg[:, :HALF]
    msg_ref[1] = msg[:, HALF:]


def _update(h, agg, wn, bn):
    n = N_NODES // R_BLK
    return pl.pallas_call(
        _update_body,
        grid=(n,),
        in_specs=[
            pl.BlockSpec((R_BLK, N_FEAT), lambda j: (j, 0)),
            pl.BlockSpec((2, R_BLK, HALF), lambda j: (0, j, 0)),
            pl.BlockSpec((N_FEAT, N_FEAT), lambda j: (0, 0)),
            pl.BlockSpec((1, N_FEAT), lambda j: (0, 0)),
        ],
        out_specs=[
            pl.BlockSpec((R_BLK, N_FEAT), lambda j: (j, 0)),
            pl.BlockSpec((2, R_BLK, HALF), lambda j: (0, j, 0)),
        ],
        out_shape=[
            jax.ShapeDtypeStruct((N_NODES, N_FEAT), jnp.float32),
            jax.ShapeDtypeStruct((2, N_NODES, HALF), jnp.float32),
        ],
    )(h, agg, wn, bn)


def _pool_body(h_ref, agg_ref, b_ref, fcw_ref, fcb_ref,
               w1b_ref, b1b_ref, w2b_ref, b2b_ref,
               w1e_ref, b1e_ref, w2e_ref, b2e_ref,
               obg_ref, oeh_ref, acc_ref):
    j = pl.program_id(0)
    nsteps = pl.num_programs(0)

    @pl.when(j == 0)
    def _():
        acc_ref[...] = jnp.zeros_like(acc_ref)

    agg = jnp.concatenate([agg_ref[0], agg_ref[1]], axis=1)
    hn = _softplus(h_ref[...] + agg)
    haug = jnp.concatenate(
        [hn, jnp.ones((P_BLK, 8), jnp.float32)], axis=1)
    oh = (lax.broadcasted_iota(jnp.int32, (P_BLK, N_GRAPHS), 1) == b_ref[...]
          ).astype(jnp.float32)
    acc_ref[...] += lax.dot_general(oh, haug, (((0,), (0,)), ((), ())),
                                    preferred_element_type=jnp.float32, precision=lax.Precision.HIGHEST)

    @pl.when(j == nsteps - 1)
    def _():
        acc = acc_ref[...]
        c = acc[:, :N_FEAT] / jnp.maximum(acc[:, N_FEAT:N_FEAT + 1], 1.0)
        cc = jnp.maximum(
            jnp.dot(c, fcw_ref[...], preferred_element_type=jnp.float32,
                    precision=lax.Precision.DEFAULT)
            + fcb_ref[...], 0.0)
        hb = jnp.maximum(
            jnp.dot(cc, w1b_ref[...], preferred_element_type=jnp.float32,
                    precision=lax.Precision.DEFAULT)
            + b1b_ref[...], 0.0)
        obg_ref[...] = jnp.dot(
            hb, w2b_ref[...], preferred_element_type=jnp.float32,
            precision=lax.Precision.DEFAULT) + b2b_ref[...]
        he = jnp.maximum(
            jnp.dot(cc, w1e_ref[...], preferred_element_type=jnp.float32,
                    precision=lax.Precision.DEFAULT)
            + b1e_ref[...], 0.0)
        oeh_ref[...] = jnp.dot(
            he, w2e_ref[...], preferred_element_type=jnp.float32,
            precision=lax.Precision.DEFAULT) + b2e_ref[...]


def _pool_heads(h, agg, bc, fcw, fcb, hb, he):
    n = N_NODES // P_BLK
    full = lambda shape: pl.BlockSpec(shape, lambda j: tuple(0 for _ in shape))
    return pl.pallas_call(
        _pool_body,
        grid=(n,),
        in_specs=[
            pl.BlockSpec((P_BLK, N_FEAT), lambda j: (j, 0)),
            pl.BlockSpec((2, P_BLK, HALF), lambda j: (0, j, 0)),
            pl.BlockSpec((P_BLK, 1), lambda j: (j, 0)),
            full((N_FEAT, 2 * N_FEAT)), full((1, 2 * N_FEAT)),
            full((2 * N_FEAT, N_FEAT)), full((1, N_FEAT)),
            full((N_FEAT, 1)), full((1, 1)),
            full((2 * N_FEAT, N_FEAT)), full((1, N_FEAT)),
            full((N_FEAT, 1)), full((1, 1)),
        ],
        out_specs=[
            pl.BlockSpec((N_GRAPHS, 1), lambda j: (0, 0)),
            pl.BlockSpec((N_GRAPHS, 1), lambda j: (0, 0)),
        ],
        out_shape=[
            jax.ShapeDtypeStruct((N_GRAPHS, 1), jnp.float32),
            jax.ShapeDtypeStruct((N_GRAPHS, 1), jnp.float32),
        ],
        scratch_shapes=[pltpu.VMEM((N_GRAPHS, N_FEAT + 8), jnp.float32)],
    )(h, agg, bc, fcw, fcb,
      hb['W1'], hb['b1'].reshape(1, -1), hb['W2'], hb['b2'].reshape(1, 1),
      he['W1'], he['b1'].reshape(1, -1), he['W2'], he['b2'].reshape(1, 1))


def _sc_block_body(blk, src_hbm, dst_hbm, msg_hbm, gate_hbm, agg_hbm,
                   sidx, didx, grow, gatev, accum, sem_a, sem_g):
    c = lax.axis_index("c")
    s = lax.axis_index("s")
    coff = (c * N_NODES).astype(jnp.int32)

    zeros16 = jnp.zeros((16,), jnp.float32)

    def _zb(e, carry):
        grow[e, pl.ds(0, 16)] = zeros16
        grow[e, pl.ds(16, 16)] = zeros16
        return carry
    lax.fori_loop(0, CH * ROW_W, _zb, 0, unroll=4)
    zsz = CH * ROW_W
    for t in range(ACC_STRIPE // zsz):
        pltpu.sync_copy(grow.at[pl.ds(0, zsz)],
                        accum.at[pl.ds(s * ACC_STRIPE + t * zsz, zsz)])
    rem = ACC_STRIPE % zsz
    if rem:
        pltpu.sync_copy(
            grow.at[pl.ds(0, rem)],
            accum.at[pl.ds(s * ACC_STRIPE + ACC_STRIPE - rem, rem)])
    plsc.subcore_barrier()

    def _process(rb, nrows):
        pltpu.sync_copy(src_hbm.at[pl.ds(rb, nrows)], sidx.at[pl.ds(0, nrows)])
        pltpu.sync_copy(dst_hbm.at[pl.ds(rb, nrows)], didx.at[pl.ds(0, nrows)])
        def _adj(r, carry):
            for q in range(ROW_W // 16):
                sl = pl.ds(q * 16, 16)
                sidx[r, sl] = sidx[r, sl] + coff
            return carry
        lax.fori_loop(0, nrows, _adj, 0)
        gd = pltpu.async_copy(
            gate_hbm.at[blk, pl.ds(c * E_PAD + rb * ROW_W, nrows * ROW_W)],
            gatev.at[pl.ds(0, nrows * ROW_W)], sem_g)
        gds = []
        for k in range(nrows):
            gds.append(pltpu.async_copy(
                msg_hbm.at[sidx.at[k]],
                grow.at[pl.ds(k * ROW_W, ROW_W)], sem_a))
        gd.wait()
        for d in gds:
            d.wait()
        def _mul(e, carry):
            for q in range(HALF // 16):
                sl = pl.ds(q * 16, 16)
                grow[e, sl] = grow[e, sl] * gatev[e, sl]
            return carry
        lax.fori_loop(0, nrows * ROW_W, _mul, 0, unroll=8)
        for k in range(nrows):
            pltpu.sync_copy(grow.at[pl.ds(k * ROW_W, ROW_W)],
                            accum.at[didx.at[k]], add=True)

    def _chunk(j, carry):
        _process(s * ROWS_PER_SUB + j * CH, CH)
        return carry
    lax.fori_loop(0, N_CHUNKS, _chunk, 0)

    plsc.subcore_barrier()
    pltpu.sync_copy(
        accum.at[pl.ds(s * ACC_STRIPE, ACC_STRIPE)],
        agg_hbm.at[pl.ds(c * ACC_ROWS + s * ACC_STRIPE, ACC_STRIPE)])


def _sc_block(blk):
    mesh = plsc.VectorSubcoreMesh(core_axis_name="c", subcore_axis_name="s")
    return pl.kernel(
        functools.partial(_sc_block_body, blk),
        out_type=jax.ShapeDtypeStruct((2 * ACC_ROWS, HALF), jnp.float32),
        mesh=mesh,
        scratch_types=[
            pltpu.VMEM((CH, ROW_W), jnp.int32),
            pltpu.VMEM((CH, ROW_W), jnp.int32),
            pltpu.VMEM((CH * ROW_W, HALF), jnp.float32),
            pltpu.VMEM((CH * ROW_W, HALF), jnp.float32),
            pltpu.VMEM_SHARED((ACC_ROWS, HALF), jnp.float32),
            pltpu.SemaphoreType.DMA,
            pltpu.SemaphoreType.DMA,
        ],
        compiler_params=pltpu.CompilerParams(use_tc_tiling_on_sc=False),
        name=f"mp_block{blk}",
    )


def kernel(x, edge_index, edge_attr, batch, emb, params):
    npad = E_PAD - N_EDGES
    src2d = jnp.concatenate(
        [edge_index[0].astype(jnp.int32), jnp.zeros((npad,), jnp.int32)]
    ).reshape(N_ROWS, ROW_W)
    dst2d = jnp.concatenate(
        [edge_index[1].astype(jnp.int32),
         jnp.full((npad,), N_NODES, jnp.int32)]
    ).reshape(N_ROWS, ROW_W)
    dcol = jnp.concatenate(
        [edge_attr.astype(jnp.float32), jnp.zeros((npad,), jnp.float32)]
    ).reshape(E_PAD, 1)
    xcol = x.astype(jnp.int32).reshape(N_NODES, 1)
    bcol = batch.astype(jnp.int32).reshape(N_NODES, 1)

    embp = jnp.zeros((128, N_FEAT), jnp.float32).at[:emb.shape[0]].set(emb)
    wep = jnp.stack([
        jnp.zeros((128, N_FEAT), jnp.float32)
        .at[:N_RBF].set(params['block' + str(i)]['We'])
        for i in range(N_CONV)])
    bes = jnp.stack([params['block' + str(i)]['be'] for i in range(N_CONV)])

    gates = _gates(dcol, wep, bes).reshape(N_CONV, 2 * E_PAD, HALF)

    h, msg = _embed(xcol, embp, params['block0']['Wn'],
                    params['block0']['bn'].reshape(1, N_FEAT))

    agg = None
    for i in range(N_CONV):
        aggf = _sc_block(i)(src2d, dst2d, msg.reshape(2 * N_NODES, HALF),
                            gates)
        agg = aggf.reshape(2, ACC_ROWS, HALF)
        if i + 1 < N_CONV:
            p = params['block' + str(i + 1)]
            h, msg = _update(h, agg, p['Wn'], p['bn'].reshape(1, N_FEAT))

    out_bg, out_eh = _pool_heads(h, agg, bcol, params['fc_W'],
                                 params['fc_b'].reshape(1, -1),
                                 params['head_bg'], params['head_eh'])
    return (out_bg, out_eh)


# speedup vs baseline: 2.3492x; 1.4122x over previous
import functools

import jax
import jax.numpy as jnp
from jax import lax
from jax.experimental import pallas as pl
from jax.experimental.pallas import tpu as pltpu
from jax.experimental.pallas import tpu_sc as plsc

N_FEAT = 64
HALF = 32
N_RBF = 10
N_CONV = 3
N_NODES = 50000
N_EDGES = 800000
N_GRAPHS = 256

ROW_W = 128
E_PAD = 819200
N_ROWS = E_PAD // ROW_W
ROWS_PER_SUB = N_ROWS // 16
CH = 2
N_CHUNKS = ROWS_PER_SUB // CH
ACC_STRIPE = 3128
ACC_ROWS = 16 * ACC_STRIPE

E_BLK = 3200
R_BLK = 2000
P_BLK = 2000


def _softplus(v):
    return jnp.maximum(v, 0.0) + jnp.log(1.0 + jnp.exp(-jnp.abs(v)))


def _gate_body(d_ref, wep_ref, bes_ref, out_ref):
    step = 6.0 / (N_RBF - 1)
    coeff = -0.5 / step**2
    q = E_BLK // 4
    lane = lax.broadcasted_iota(jnp.int32, (q, 128), 1).astype(jnp.float32)
    offs = lane * step
    parts = [[[None] * 4 for _ in range(2)] for _ in range(N_CONV)]
    for u in range(4):
        z = d_ref[:, u:u + 1] - offs
        ea = jnp.exp(coeff * z * z)
        for i in range(N_CONV):
            g = lax.dot_general(ea, wep_ref[i], (((1,), (0,)), ((), ())),
                                preferred_element_type=jnp.float32,
                                precision=lax.Precision.DEFAULT)
            g = g + bes_ref[i:i + 1, :]
            parts[i][0][u] = g[:, :HALF]
            parts[i][1][u] = g[:, HALF:]
    for i in range(N_CONV):
        for h in range(2):
            out_ref[i, h] = jnp.concatenate(parts[i][h], axis=1)


def _gates(d, wep, bes):
    n = E_PAD // E_BLK
    return pl.pallas_call(
        _gate_body,
        grid=(n,),
        in_specs=[
            pl.BlockSpec((E_BLK // 4, 4), lambda j: (j, 0)),
            pl.BlockSpec((N_CONV, 128, N_FEAT), lambda j: (0, 0, 0)),
            pl.BlockSpec((N_CONV, N_FEAT), lambda j: (0, 0)),
        ],
        out_specs=pl.BlockSpec((N_CONV, 2, E_BLK // 4, 128),
                               lambda j: (0, 0, j, 0)),
        out_shape=jax.ShapeDtypeStruct((N_CONV, 2, E_PAD // 4, 128),
                                       jnp.float32),
    )(d, wep, bes)


def _embed_body(x_ref, embp_ref, wn_ref, bn_ref, h_ref, msg_ref):
    xv = x_ref[...]
    oh = (lax.broadcasted_iota(jnp.int32, (R_BLK, 128), 1) == xv
          ).astype(jnp.float32)
    h = jnp.dot(oh, embp_ref[...], preferred_element_type=jnp.float32, precision=lax.Precision.HIGHEST)
    msg = jnp.dot(h, wn_ref[...], preferred_element_type=jnp.float32,
                  precision=lax.Precision.DEFAULT) + bn_ref[...]
    h_ref[...] = h
    msg_ref[0] = msg[:, :HALF]
    msg_ref[1] = msg[:, HALF:]


def _embed(xc, embp, wn, bn):
    n = N_NODES // R_BLK
    return pl.pallas_call(
        _embed_body,
        grid=(n,),
        in_specs=[
            pl.BlockSpec((R_BLK, 1), lambda j: (j, 0)),
            pl.BlockSpec((128, N_FEAT), lambda j: (0, 0)),
            pl.BlockSpec((N_FEAT, N_FEAT), lambda j: (0, 0)),
            pl.BlockSpec((1, N_FEAT), lambda j: (0, 0)),
        ],
        out_specs=[
            pl.BlockSpec((R_BLK, N_FEAT), lambda j: (j, 0)),
            pl.BlockSpec((2, R_BLK, HALF), lambda j: (0, j, 0)),
        ],
        out_shape=[
            jax.ShapeDtypeStruct((N_NODES, N_FEAT), jnp.float32),
            jax.ShapeDtypeStruct((2, N_NODES, HALF), jnp.float32),
        ],
    )(xc, embp, wn, bn)


def _update_body(h_ref, agg_ref, wn_ref, bn_ref, h_out_ref, msg_ref):
    agg = jnp.concatenate([agg_ref[0], agg_ref[1]], axis=1)
    hn = _softplus(h_ref[...] + agg)
    msg = jnp.dot(hn, wn_ref[...], preferred_element_type=jnp.float32,
                  precision=lax.Precision.DEFAULT) + bn_ref[...]
    h_out_ref[...] = hn
    msg_ref[0] = msg[:, :HALF]
    msg_ref[1] = msg[:, HALF:]


def _update(h, agg, wn, bn):
    n = N_NODES // R_BLK
    return pl.pallas_call(
        _update_body,
        grid=(n,),
        in_specs=[
            pl.BlockSpec((R_BLK, N_FEAT), lambda j: (j, 0)),
            pl.BlockSpec((2, R_BLK, HALF), lambda j: (0, j, 0)),
            pl.BlockSpec((N_FEAT, N_FEAT), lambda j: (0, 0)),
            pl.BlockSpec((1, N_FEAT), lambda j: (0, 0)),
        ],
        out_specs=[
            pl.BlockSpec((R_BLK, N_FEAT), lambda j: (j, 0)),
            pl.BlockSpec((2, R_BLK, HALF), lambda j: (0, j, 0)),
        ],
        out_shape=[
            jax.ShapeDtypeStruct((N_NODES, N_FEAT), jnp.float32),
            jax.ShapeDtypeStruct((2, N_NODES, HALF), jnp.float32),
        ],
    )(h, agg, wn, bn)


def _pool_body(h_ref, agg_ref, b_ref, fcw_ref, fcb_ref,
               w1b_ref, b1b_ref, w2b_ref, b2b_ref,
               w1e_ref, b1e_ref, w2e_ref, b2e_ref,
               obg_ref, oeh_ref, acc_ref):
    j = pl.program_id(0)
    nsteps = pl.num_programs(0)

    @pl.when(j == 0)
    def _():
        acc_ref[...] = jnp.zeros_like(acc_ref)

    agg = jnp.concatenate([agg_ref[0], agg_ref[1]], axis=1)
    hn = _softplus(h_ref[...] + agg)
    haug = jnp.concatenate(
        [hn, jnp.ones((P_BLK, 8), jnp.float32)], axis=1)
    oh = (lax.broadcasted_iota(jnp.int32, (P_BLK, N_GRAPHS), 1) == b_ref[...]
          ).astype(jnp.float32)
    acc_ref[...] += lax.dot_general(oh, haug, (((0,), (0,)), ((), ())),
                                    preferred_element_type=jnp.float32, precision=lax.Precision.HIGHEST)

    @pl.when(j == nsteps - 1)
    def _():
        acc = acc_ref[...]
        c = acc[:, :N_FEAT] / jnp.maximum(acc[:, N_FEAT:N_FEAT + 1], 1.0)
        cc = jnp.maximum(
            jnp.dot(c, fcw_ref[...], preferred_element_type=jnp.float32,
                    precision=lax.Precision.DEFAULT)
            + fcb_ref[...], 0.0)
        hb = jnp.maximum(
            jnp.dot(cc, w1b_ref[...], preferred_element_type=jnp.float32,
                    precision=lax.Precision.DEFAULT)
            + b1b_ref[...], 0.0)
        obg_ref[...] = jnp.dot(
            hb, w2b_ref[...], preferred_element_type=jnp.float32,
            precision=lax.Precision.DEFAULT) + b2b_ref[...]
        he = jnp.maximum(
            jnp.dot(cc, w1e_ref[...], preferred_element_type=jnp.float32,
                    precision=lax.Precision.DEFAULT)
            + b1e_ref[...], 0.0)
        oeh_ref[...] = jnp.dot(
            he, w2e_ref[...], preferred_element_type=jnp.float32,
            precision=lax.Precision.DEFAULT) + b2e_ref[...]


def _pool_heads(h, agg, bc, fcw, fcb, hb, he):
    n = N_NODES // P_BLK
    full = lambda shape: pl.BlockSpec(shape, lambda j: tuple(0 for _ in shape))
    return pl.pallas_call(
        _pool_body,
        grid=(n,),
        in_specs=[
            pl.BlockSpec((P_BLK, N_FEAT), lambda j: (j, 0)),
            pl.BlockSpec((2, P_BLK, HALF), lambda j: (0, j, 0)),
            pl.BlockSpec((P_BLK, 1), lambda j: (j, 0)),
            full((N_FEAT, 2 * N_FEAT)), full((1, 2 * N_FEAT)),
            full((2 * N_FEAT, N_FEAT)), full((1, N_FEAT)),
            full((N_FEAT, 1)), full((1, 1)),
            full((2 * N_FEAT, N_FEAT)), full((1, N_FEAT)),
            full((N_FEAT, 1)), full((1, 1)),
        ],
        out_specs=[
            pl.BlockSpec((N_GRAPHS, 1), lambda j: (0, 0)),
            pl.BlockSpec((N_GRAPHS, 1), lambda j: (0, 0)),
        ],
        out_shape=[
            jax.ShapeDtypeStruct((N_GRAPHS, 1), jnp.float32),
            jax.ShapeDtypeStruct((N_GRAPHS, 1), jnp.float32),
        ],
        scratch_shapes=[pltpu.VMEM((N_GRAPHS, N_FEAT + 8), jnp.float32)],
    )(h, agg, bc, fcw, fcb,
      hb['W1'], hb['b1'].reshape(1, -1), hb['W2'], hb['b2'].reshape(1, 1),
      he['W1'], he['b1'].reshape(1, -1), he['W2'], he['b2'].reshape(1, 1))


def _sc_block_body(blk, src_hbm, dst_hbm, msg_hbm, gate_hbm, agg_hbm,
                   sidx, didx, grow, gatev, accum, sem_a, sem_g):
    c = lax.axis_index("c")
    s = lax.axis_index("s")
    coff = (c * N_NODES).astype(jnp.int32)

    zeros16 = jnp.zeros((16,), jnp.float32)

    def _zb(e, carry):
        grow[e, pl.ds(0, 16)] = zeros16
        grow[e, pl.ds(16, 16)] = zeros16
        return carry
    lax.fori_loop(0, CH * ROW_W, _zb, 0, unroll=4)
    zsz = CH * ROW_W
    for t in range(ACC_STRIPE // zsz):
        pltpu.sync_copy(grow.at[pl.ds(0, zsz)],
                        accum.at[pl.ds(s * ACC_STRIPE + t * zsz, zsz)])
    rem = ACC_STRIPE % zsz
    if rem:
        pltpu.sync_copy(
            grow.at[pl.ds(0, rem)],
            accum.at[pl.ds(s * ACC_STRIPE + ACC_STRIPE - rem, rem)])
    plsc.subcore_barrier()

    def _process(rb, nrows):
        pltpu.sync_copy(src_hbm.at[pl.ds(rb, nrows)], sidx.at[pl.ds(0, nrows)])
        pltpu.sync_copy(dst_hbm.at[pl.ds(rb, nrows)], didx.at[pl.ds(0, nrows)])
        def _adj(r, carry):
            for q in range(ROW_W // 16):
                sl = pl.ds(q * 16, 16)
                sidx[r, sl] = sidx[r, sl] + coff
            return carry
        lax.fori_loop(0, nrows, _adj, 0)
        grows = nrows * (ROW_W // 4)
        gd = pltpu.async_copy(
            gate_hbm.at[blk, pl.ds(c * (E_PAD // 4) + rb * (ROW_W // 4),
                                   grows)],
            gatev.at[pl.ds(0, grows)], sem_g)
        gds = []
        for k in range(nrows):
            gds.append(pltpu.async_copy(
                msg_hbm.at[sidx.at[k]],
                grow.at[pl.ds(k * ROW_W, ROW_W)], sem_a))
        gd.wait()
        for d in gds:
            d.wait()
        def _mul(t, carry):
            for u in range(4):
                for q in range(HALF // 16):
                    slm = pl.ds(q * 16, 16)
                    slg = pl.ds(u * HALF + q * 16, 16)
                    grow[4 * t + u, slm] = grow[4 * t + u, slm] * gatev[t, slg]
            return carry
        lax.fori_loop(0, grows, _mul, 0, unroll=4)
        for k in range(nrows):
            pltpu.sync_copy(grow.at[pl.ds(k * ROW_W, ROW_W)],
                            accum.at[didx.at[k]], add=True)

    def _chunk(j, carry):
        _process(s * ROWS_PER_SUB + j * CH, CH)
        return carry
    lax.fori_loop(0, N_CHUNKS, _chunk, 0)

    plsc.subcore_barrier()
    pltpu.sync_copy(
        accum.at[pl.ds(s * ACC_STRIPE, ACC_STRIPE)],
        agg_hbm.at[pl.ds(c * ACC_ROWS + s * ACC_STRIPE, ACC_STRIPE)])


def _sc_block(blk):
    mesh = plsc.VectorSubcoreMesh(core_axis_name="c", subcore_axis_name="s")
    return pl.kernel(
        functools.partial(_sc_block_body, blk),
        out_type=jax.ShapeDtypeStruct((2 * ACC_ROWS, HALF), jnp.float32),
        mesh=mesh,
        scratch_types=[
            pltpu.VMEM((CH, ROW_W), jnp.int32),
            pltpu.VMEM((CH, ROW_W), jnp.int32),
            pltpu.VMEM((CH * ROW_W, HALF), jnp.float32),
            pltpu.VMEM((CH * ROW_W // 4, 128), jnp.float32),
            pltpu.VMEM_SHARED((ACC_ROWS, HALF), jnp.float32),
            pltpu.SemaphoreType.DMA,
            pltpu.SemaphoreType.DMA,
        ],
        compiler_params=pltpu.CompilerParams(use_tc_tiling_on_sc=False),
        name=f"mp_block{blk}",
    )


def kernel(x, edge_index, edge_attr, batch, emb, params):
    npad = E_PAD - N_EDGES
    src2d = jnp.concatenate(
        [edge_index[0].astype(jnp.int32), jnp.zeros((npad,), jnp.int32)]
    ).reshape(N_ROWS, ROW_W)
    dst2d = jnp.concatenate(
        [edge_index[1].astype(jnp.int32),
         jnp.full((npad,), N_NODES, jnp.int32)]
    ).reshape(N_ROWS, ROW_W)
    dcol = jnp.concatenate(
        [edge_attr.astype(jnp.float32), jnp.zeros((npad,), jnp.float32)]
    ).reshape(E_PAD // 4, 4)
    xcol = x.astype(jnp.int32).reshape(N_NODES, 1)
    bcol = batch.astype(jnp.int32).reshape(N_NODES, 1)

    embp = jnp.zeros((128, N_FEAT), jnp.float32).at[:emb.shape[0]].set(emb)
    wep = jnp.stack([
        jnp.zeros((128, N_FEAT), jnp.float32)
        .at[:N_RBF].set(params['block' + str(i)]['We'])
        for i in range(N_CONV)])
    bes = jnp.stack([params['block' + str(i)]['be'] for i in range(N_CONV)])

    gates = _gates(dcol, wep, bes).reshape(N_CONV, 2 * (E_PAD // 4), 128)

    h, msg = _embed(xcol, embp, params['block0']['Wn'],
                    params['block0']['bn'].reshape(1, N_FEAT))

    agg = None
    for i in range(N_CONV):
        aggf = _sc_block(i)(src2d, dst2d, msg.reshape(2 * N_NODES, HALF),
                            gates)
        agg = aggf.reshape(2, ACC_ROWS, HALF)
        if i + 1 < N_CONV:
            p = params['block' + str(i + 1)]
            h, msg = _update(h, agg, p['Wn'], p['bn'].reshape(1, N_FEAT))

    out_bg, out_eh = _pool_heads(h, agg, bcol, params['fc_W'],
                                 params['fc_b'].reshape(1, -1),
                                 params['head_bg'], params['head_eh'])
    return (out_bg, out_eh)


# speedup vs baseline: 2.3496x; 1.0002x over previous
import functools

import jax
import jax.numpy as jnp
from jax import lax
from jax.experimental import pallas as pl
from jax.experimental.pallas import tpu as pltpu
from jax.experimental.pallas import tpu_sc as plsc

N_FEAT = 64
HALF = 32
N_RBF = 10
N_CONV = 3
N_NODES = 50000
N_EDGES = 800000
N_GRAPHS = 256

ROW_W = 128
E_PAD = 819200
N_ROWS = E_PAD // ROW_W
ROWS_PER_SUB = N_ROWS // 16
CH = 2
N_CHUNKS = ROWS_PER_SUB // CH
ACC_STRIPE = 3128
ACC_ROWS = 16 * ACC_STRIPE

E_BLK = 3200
R_BLK = 2000
P_BLK = 2000


def _softplus(v):
    return jnp.maximum(v, 0.0) + jnp.log(1.0 + jnp.exp(-jnp.abs(v)))


def _gate_body(d_ref, wep_ref, bes_ref, out_ref):
    step = 6.0 / (N_RBF - 1)
    coeff = -0.5 / step**2
    q = E_BLK // 4
    lane = lax.broadcasted_iota(jnp.int32, (q, 128), 1).astype(jnp.float32)
    offs = lane * step
    parts = [[[None] * 4 for _ in range(2)] for _ in range(N_CONV)]
    for u in range(4):
        z = d_ref[:, u:u + 1] - offs
        ea = jnp.exp(coeff * z * z)
        for i in range(N_CONV):
            g = lax.dot_general(ea, wep_ref[i], (((1,), (0,)), ((), ())),
                                preferred_element_type=jnp.float32,
                                precision=lax.Precision.DEFAULT)
            g = g + bes_ref[i:i + 1, :]
            parts[i][0][u] = g[:, :HALF]
            parts[i][1][u] = g[:, HALF:]
    for i in range(N_CONV):
        for h in range(2):
            out_ref[i, h] = jnp.concatenate(parts[i][h], axis=1)


def _gates(d, wep, bes):
    n = E_PAD // E_BLK
    return pl.pallas_call(
        _gate_body,
        grid=(n,),
        in_specs=[
            pl.BlockSpec((E_BLK // 4, 4), lambda j: (j, 0)),
            pl.BlockSpec((N_CONV, 128, N_FEAT), lambda j: (0, 0, 0)),
            pl.BlockSpec((N_CONV, N_FEAT), lambda j: (0, 0)),
        ],
        out_specs=pl.BlockSpec((N_CONV, 2, E_BLK // 4, 128),
                               lambda j: (0, 0, j, 0)),
        out_shape=jax.ShapeDtypeStruct((N_CONV, 2, E_PAD // 4, 128),
                                       jnp.float32),
    )(d, wep, bes)


def _embed_body(x_ref, embp_ref, wn_ref, bn_ref, h_ref, msg_ref):
    xv = x_ref[...]
    oh = (lax.broadcasted_iota(jnp.int32, (R_BLK, 128), 1) == xv
          ).astype(jnp.float32)
    h = jnp.dot(oh, embp_ref[...], preferred_element_type=jnp.float32, precision=lax.Precision.HIGHEST)
    msg = jnp.dot(h, wn_ref[...], preferred_element_type=jnp.float32,
                  precision=lax.Precision.DEFAULT) + bn_ref[...]
    h_ref[...] = h
    msg_ref[0] = msg[:, :HALF]
    msg_ref[1] = msg[:, HALF:]


def _embed(xc, embp, wn, bn):
    n = N_NODES // R_BLK
    return pl.pallas_call(
        _embed_body,
        grid=(n,),
        in_specs=[
            pl.BlockSpec((R_BLK, 1), lambda j: (j, 0)),
            pl.BlockSpec((128, N_FEAT), lambda j: (0, 0)),
            pl.BlockSpec((N_FEAT, N_FEAT), lambda j: (0, 0)),
            pl.BlockSpec((1, N_FEAT), lambda j: (0, 0)),
        ],
        out_specs=[
            pl.BlockSpec((R_BLK, N_FEAT), lambda j: (j, 0)),
            pl.BlockSpec((2, R_BLK, HALF), lambda j: (0, j, 0)),
        ],
        out_shape=[
            jax.ShapeDtypeStruct((N_NODES, N_FEAT), jnp.float32),
            jax.ShapeDtypeStruct((2, N_NODES, HALF), jnp.float32),
        ],
    )(xc, embp, wn, bn)


def _update_body(h_ref, agg_ref, wn_ref, bn_ref, h_out_ref, msg_ref):
    agg = jnp.concatenate([agg_ref[0], agg_ref[1]], axis=1)
    hn = _softplus(h_ref[...] + agg)
    msg = jnp.dot(hn, wn_ref[...], preferred_element_type=jnp.float32,
                  precision=lax.Precision.DEFAULT) + bn_ref[...]
    h_out_ref[...] = hn
    msg_ref[0] = msg[:, :HALF]
    msg_ref[1] = msg[:, HALF:]


def _update(h, agg, wn, bn):
    n = N_NODES // R_BLK
    return pl.pallas_call(
        _update_body,
        grid=(n,),
        in_specs=[
            pl.BlockSpec((R_BLK, N_FEAT), lambda j: (j, 0)),
            pl.BlockSpec((2, R_BLK, HALF), lambda j: (0, j, 0)),
            pl.BlockSpec((N_FEAT, N_FEAT), lambda j: (0, 0)),
            pl.BlockSpec((1, N_FEAT), lambda j: (0, 0)),
        ],
        out_specs=[
            pl.BlockSpec((R_BLK, N_FEAT), lambda j: (j, 0)),
            pl.BlockSpec((2, R_BLK, HALF), lambda j: (0, j, 0)),
        ],
        out_shape=[
            jax.ShapeDtypeStruct((N_NODES, N_FEAT), jnp.float32),
            jax.ShapeDtypeStruct((2, N_NODES, HALF), jnp.float32),
        ],
    )(h, agg, wn, bn)


def _pool_body(h_ref, agg_ref, b_ref, fcw_ref, fcb_ref,
               w1b_ref, b1b_ref, w2b_ref, b2b_ref,
               w1e_ref, b1e_ref, w2e_ref, b2e_ref,
               obg_ref, oeh_ref, acc_ref):
    j = pl.program_id(0)
    nsteps = pl.num_programs(0)

    @pl.when(j == 0)
    def _():
        acc_ref[...] = jnp.zeros_like(acc_ref)

    agg = jnp.concatenate([agg_ref[0], agg_ref[1]], axis=1)
    hn = _softplus(h_ref[...] + agg)
    haug = jnp.concatenate(
        [hn, jnp.ones((P_BLK, 8), jnp.float32)], axis=1)
    oh = (lax.broadcasted_iota(jnp.int32, (P_BLK, N_GRAPHS), 1) == b_ref[...]
          ).astype(jnp.float32)
    acc_ref[...] += lax.dot_general(oh, haug, (((0,), (0,)), ((), ())),
                                    preferred_element_type=jnp.float32, precision=lax.Precision.HIGHEST)

    @pl.when(j == nsteps - 1)
    def _():
        acc = acc_ref[...]
        c = acc[:, :N_FEAT] / jnp.maximum(acc[:, N_FEAT:N_FEAT + 1], 1.0)
        cc = jnp.maximum(
            jnp.dot(c, fcw_ref[...], preferred_element_type=jnp.float32,
                    precision=lax.Precision.DEFAULT)
            + fcb_ref[...], 0.0)
        hb = jnp.maximum(
            jnp.dot(cc, w1b_ref[...], preferred_element_type=jnp.float32,
                    precision=lax.Precision.DEFAULT)
            + b1b_ref[...], 0.0)
        obg_ref[...] = jnp.dot(
            hb, w2b_ref[...], preferred_element_type=jnp.float32,
            precision=lax.Precision.DEFAULT) + b2b_ref[...]
        he = jnp.maximum(
            jnp.dot(cc, w1e_ref[...], preferred_element_type=jnp.float32,
                    precision=lax.Precision.DEFAULT)
            + b1e_ref[...], 0.0)
        oeh_ref[...] = jnp.dot(
            he, w2e_ref[...], preferred_element_type=jnp.float32,
            precision=lax.Precision.DEFAULT) + b2e_ref[...]


def _pool_heads(h, agg, bc, fcw, fcb, hb, he):
    n = N_NODES // P_BLK
    full = lambda shape: pl.BlockSpec(shape, lambda j: tuple(0 for _ in shape))
    return pl.pallas_call(
        _pool_body,
        grid=(n,),
        in_specs=[
            pl.BlockSpec((P_BLK, N_FEAT), lambda j: (j, 0)),
            pl.BlockSpec((2, P_BLK, HALF), lambda j: (0, j, 0)),
            pl.BlockSpec((P_BLK, 1), lambda j: (j, 0)),
            full((N_FEAT, 2 * N_FEAT)), full((1, 2 * N_FEAT)),
            full((2 * N_FEAT, N_FEAT)), full((1, N_FEAT)),
            full((N_FEAT, 1)), full((1, 1)),
            full((2 * N_FEAT, N_FEAT)), full((1, N_FEAT)),
            full((N_FEAT, 1)), full((1, 1)),
        ],
        out_specs=[
            pl.BlockSpec((N_GRAPHS, 1), lambda j: (0, 0)),
            pl.BlockSpec((N_GRAPHS, 1), lambda j: (0, 0)),
        ],
        out_shape=[
            jax.ShapeDtypeStruct((N_GRAPHS, 1), jnp.float32),
            jax.ShapeDtypeStruct((N_GRAPHS, 1), jnp.float32),
        ],
        scratch_shapes=[pltpu.VMEM((N_GRAPHS, N_FEAT + 8), jnp.float32)],
    )(h, agg, bc, fcw, fcb,
      hb['W1'], hb['b1'].reshape(1, -1), hb['W2'], hb['b2'].reshape(1, 1),
      he['W1'], he['b1'].reshape(1, -1), he['W2'], he['b2'].reshape(1, 1))


def _sc_block_body(blk, src_hbm, dst_hbm, msg_hbm, gate_hbm, agg_hbm,
                   sidx, didx, grow, gatev, accum, sem_a, sem_g):
    c = lax.axis_index("c")
    s = lax.axis_index("s")
    coff = (c * N_NODES).astype(jnp.int32)

    zeros16 = jnp.zeros((16,), jnp.float32)

    def _zb(e, carry):
        grow[e, pl.ds(0, 16)] = zeros16
        grow[e, pl.ds(16, 16)] = zeros16
        return carry
    lax.fori_loop(0, CH * ROW_W, _zb, 0, unroll=4)
    zsz = CH * ROW_W
    for t in range(ACC_STRIPE // zsz):
        pltpu.sync_copy(grow.at[pl.ds(0, zsz)],
                        accum.at[pl.ds(s * ACC_STRIPE + t * zsz, zsz)])
    rem = ACC_STRIPE % zsz
    if rem:
        pltpu.sync_copy(
            grow.at[pl.ds(0, rem)],
            accum.at[pl.ds(s * ACC_STRIPE + ACC_STRIPE - rem, rem)])
    plsc.subcore_barrier()

    def _process(rb, nrows):
        pltpu.sync_copy(src_hbm.at[pl.ds(rb, nrows)], sidx.at[pl.ds(0, nrows)])
        pltpu.sync_copy(dst_hbm.at[pl.ds(rb, nrows)], didx.at[pl.ds(0, nrows)])
        def _adj(r, carry):
            for q in range(ROW_W // 16):
                sl = pl.ds(q * 16, 16)
                sidx[r, sl] = sidx[r, sl] + coff
            return carry
        lax.fori_loop(0, nrows, _adj, 0)
        grows = nrows * (ROW_W // 4)
        gbase = (blk * 2 * (E_PAD // 4)
                 + c * (E_PAD // 4) + rb * (ROW_W // 4)) * 128
        gd = pltpu.async_copy(
            gate_hbm.at[pl.ds(gbase, grows * 128)],
            gatev.at[pl.ds(0, grows * 128)], sem_g)
        gds = []
        for k in range(nrows):
            gds.append(pltpu.async_copy(
                msg_hbm.at[sidx.at[k]],
                grow.at[pl.ds(k * ROW_W, ROW_W)], sem_a))
        gd.wait()
        for d in gds:
            d.wait()
        def _mul(t, carry):
            for u in range(4):
                for q in range(HALF // 16):
                    slm = pl.ds(q * 16, 16)
                    slg = pl.ds(128 * t + u * HALF + q * 16, 16)
                    grow[4 * t + u, slm] = grow[4 * t + u, slm] * gatev[slg]
            return carry
        lax.fori_loop(0, grows, _mul, 0, unroll=4)
        for k in range(nrows):
            pltpu.sync_copy(grow.at[pl.ds(k * ROW_W, ROW_W)],
                            accum.at[didx.at[k]], add=True)

    def _chunk(j, carry):
        _process(s * ROWS_PER_SUB + j * CH, CH)
        return carry
    lax.fori_loop(0, N_CHUNKS, _chunk, 0)

    plsc.subcore_barrier()
    pltpu.sync_copy(
        accum.at[pl.ds(s * ACC_STRIPE, ACC_STRIPE)],
        agg_hbm.at[pl.ds(c * ACC_ROWS + s * ACC_STRIPE, ACC_STRIPE)])


def _sc_block(blk):
    mesh = plsc.VectorSubcoreMesh(core_axis_name="c", subcore_axis_name="s")
    return pl.kernel(
        functools.partial(_sc_block_body, blk),
        out_type=jax.ShapeDtypeStruct((2 * ACC_ROWS, HALF), jnp.float32),
        mesh=mesh,
        scratch_types=[
            pltpu.VMEM((CH, ROW_W), jnp.int32),
            pltpu.VMEM((CH, ROW_W), jnp.int32),
            pltpu.VMEM((CH * ROW_W, HALF), jnp.float32),
            pltpu.VMEM((CH * ROW_W * HALF,), jnp.float32),
            pltpu.VMEM_SHARED((ACC_ROWS, HALF), jnp.float32),
            pltpu.SemaphoreType.DMA,
            pltpu.SemaphoreType.DMA,
        ],
        compiler_params=pltpu.CompilerParams(use_tc_tiling_on_sc=False),
        name=f"mp_block{blk}",
    )


def kernel(x, edge_index, edge_attr, batch, emb, params):
    npad = E_PAD - N_EDGES
    src2d = jnp.concatenate(
        [edge_index[0].astype(jnp.int32), jnp.zeros((npad,), jnp.int32)]
    ).reshape(N_ROWS, ROW_W)
    dst2d = jnp.concatenate(
        [edge_index[1].astype(jnp.int32),
         jnp.full((npad,), N_NODES, jnp.int32)]
    ).reshape(N_ROWS, ROW_W)
    dcol = jnp.concatenate(
        [edge_attr.astype(jnp.float32), jnp.zeros((npad,), jnp.float32)]
    ).reshape(E_PAD // 4, 4)
    xcol = x.astype(jnp.int32).reshape(N_NODES, 1)
    bcol = batch.astype(jnp.int32).reshape(N_NODES, 1)

    embp = jnp.zeros((128, N_FEAT), jnp.float32).at[:emb.shape[0]].set(emb)
    wep = jnp.stack([
        jnp.zeros((128, N_FEAT), jnp.float32)
        .at[:N_RBF].set(params['block' + str(i)]['We'])
        for i in range(N_CONV)])
    bes = jnp.stack([params['block' + str(i)]['be'] for i in range(N_CONV)])

    gates = _gates(dcol, wep, bes).reshape(-1)

    h, msg = _embed(xcol, embp, params['block0']['Wn'],
                    params['block0']['bn'].reshape(1, N_FEAT))

    agg = None
    for i in range(N_CONV):
        aggf = _sc_block(i)(src2d, dst2d, msg.reshape(2 * N_NODES, HALF),
                            gates)
        agg = aggf.reshape(2, ACC_ROWS, HALF)
        if i + 1 < N_CONV:
            p = params['block' + str(i + 1)]
            h, msg = _update(h, agg, p['Wn'], p['bn'].reshape(1, N_FEAT))

    out_bg, out_eh = _pool_heads(h, agg, bcol, params['fc_W'],
                                 params['fc_b'].reshape(1, -1),
                                 params['head_bg'], params['head_eh'])
    return (out_bg, out_eh)


# speedup vs baseline: 3.2385x; 1.3783x over previous
import functools

import jax
import jax.numpy as jnp
from jax import lax
from jax.experimental import pallas as pl
from jax.experimental.pallas import tpu as pltpu
from jax.experimental.pallas import tpu_sc as plsc

N_FEAT = 64
HALF = 32
N_RBF = 10
N_CONV = 3
N_NODES = 50000
N_EDGES = 800000
N_GRAPHS = 256

ROW_W = 128
E_PAD = 819200
N_ROWS = E_PAD // ROW_W
ROWS_PER_SUB = N_ROWS // 16
CH = 2
N_CHUNKS = ROWS_PER_SUB // CH
ACC_STRIPE = 3128
ACC_ROWS = 16 * ACC_STRIPE

E_BLK = 3200
R_BLK = 2000
P_BLK = 2000


def _softplus(v):
    return jnp.maximum(v, 0.0) + jnp.log(1.0 + jnp.exp(-jnp.abs(v)))


def _gate_body(d_ref, wep_ref, bes_ref, out_ref):
    step = 6.0 / (N_RBF - 1)
    coeff = -0.5 / step**2
    q = E_BLK // 4
    lane = lax.broadcasted_iota(jnp.int32, (q, 128), 1).astype(jnp.float32)
    offs = lane * step
    parts = [[[None] * 4 for _ in range(2)] for _ in range(N_CONV)]
    for u in range(4):
        z = d_ref[:, u:u + 1] - offs
        ea = jnp.exp(coeff * z * z)
        for i in range(N_CONV):
            g = lax.dot_general(ea, wep_ref[i], (((1,), (0,)), ((), ())),
                                preferred_element_type=jnp.float32,
                                precision=lax.Precision.DEFAULT)
            g = g + bes_ref[i:i + 1, :]
            parts[i][0][u] = g[:, :HALF]
            parts[i][1][u] = g[:, HALF:]
    for i in range(N_CONV):
        for h in range(2):
            out_ref[i, h] = jnp.concatenate(parts[i][h], axis=1)


def _gates(d, wep, bes):
    n = E_PAD // E_BLK
    return pl.pallas_call(
        _gate_body,
        grid=(n,),
        in_specs=[
            pl.BlockSpec((E_BLK // 4, 4), lambda j: (j, 0)),
            pl.BlockSpec((N_CONV, 128, N_FEAT), lambda j: (0, 0, 0)),
            pl.BlockSpec((N_CONV, N_FEAT), lambda j: (0, 0)),
        ],
        out_specs=pl.BlockSpec((N_CONV, 2, E_BLK // 4, 128),
                               lambda j: (0, 0, j, 0)),
        out_shape=jax.ShapeDtypeStruct((N_CONV, 2, E_PAD // 4, 128),
                                       jnp.float32),
    )(d, wep, bes)


def _embed_body(x_ref, embp_ref, wn_ref, bn_ref, h_ref, msg_ref):
    xv = x_ref[...]
    oh = (lax.broadcasted_iota(jnp.int32, (R_BLK, 128), 1) == xv
          ).astype(jnp.float32)
    h = jnp.dot(oh, embp_ref[...], preferred_element_type=jnp.float32, precision=lax.Precision.HIGHEST)
    msg = jnp.dot(h, wn_ref[...], preferred_element_type=jnp.float32,
                  precision=lax.Precision.DEFAULT) + bn_ref[...]
    h_ref[...] = h
    msg_ref[0] = msg[:, :HALF]
    msg_ref[1] = msg[:, HALF:]


def _embed(xc, embp, wn, bn):
    n = N_NODES // R_BLK
    return pl.pallas_call(
        _embed_body,
        grid=(n,),
        in_specs=[
            pl.BlockSpec((R_BLK, 1), lambda j: (j, 0)),
            pl.BlockSpec((128, N_FEAT), lambda j: (0, 0)),
            pl.BlockSpec((N_FEAT, N_FEAT), lambda j: (0, 0)),
            pl.BlockSpec((1, N_FEAT), lambda j: (0, 0)),
        ],
        out_specs=[
            pl.BlockSpec((R_BLK, N_FEAT), lambda j: (j, 0)),
            pl.BlockSpec((2, R_BLK, HALF), lambda j: (0, j, 0)),
        ],
        out_shape=[
            jax.ShapeDtypeStruct((N_NODES, N_FEAT), jnp.float32),
            jax.ShapeDtypeStruct((2, N_NODES, HALF), jnp.float32),
        ],
    )(xc, embp, wn, bn)


def _update_body(h_ref, agg_ref, wn_ref, bn_ref, h_out_ref, msg_ref):
    agg = jnp.concatenate([agg_ref[0], agg_ref[1]], axis=1)
    hn = _softplus(h_ref[...] + agg)
    msg = jnp.dot(hn, wn_ref[...], preferred_element_type=jnp.float32,
                  precision=lax.Precision.DEFAULT) + bn_ref[...]
    h_out_ref[...] = hn
    msg_ref[0] = msg[:, :HALF]
    msg_ref[1] = msg[:, HALF:]


def _update(h, agg, wn, bn):
    n = N_NODES // R_BLK
    return pl.pallas_call(
        _update_body,
        grid=(n,),
        in_specs=[
            pl.BlockSpec((R_BLK, N_FEAT), lambda j: (j, 0)),
            pl.BlockSpec((2, R_BLK, HALF), lambda j: (0, j, 0)),
            pl.BlockSpec((N_FEAT, N_FEAT), lambda j: (0, 0)),
            pl.BlockSpec((1, N_FEAT), lambda j: (0, 0)),
        ],
        out_specs=[
            pl.BlockSpec((R_BLK, N_FEAT), lambda j: (j, 0)),
            pl.BlockSpec((2, R_BLK, HALF), lambda j: (0, j, 0)),
        ],
        out_shape=[
            jax.ShapeDtypeStruct((N_NODES, N_FEAT), jnp.float32),
            jax.ShapeDtypeStruct((2, N_NODES, HALF), jnp.float32),
        ],
    )(h, agg, wn, bn)


def _pool_body(h_ref, agg_ref, b_ref, fcw_ref, fcb_ref,
               w1b_ref, b1b_ref, w2b_ref, b2b_ref,
               w1e_ref, b1e_ref, w2e_ref, b2e_ref,
               obg_ref, oeh_ref, acc_ref):
    j = pl.program_id(0)
    nsteps = pl.num_programs(0)

    @pl.when(j == 0)
    def _():
        acc_ref[...] = jnp.zeros_like(acc_ref)

    agg = jnp.concatenate([agg_ref[0], agg_ref[1]], axis=1)
    hn = _softplus(h_ref[...] + agg)
    haug = jnp.concatenate(
        [hn, jnp.ones((P_BLK, 8), jnp.float32)], axis=1)
    oh = (lax.broadcasted_iota(jnp.int32, (P_BLK, N_GRAPHS), 1) == b_ref[...]
          ).astype(jnp.float32)
    acc_ref[...] += lax.dot_general(oh, haug, (((0,), (0,)), ((), ())),
                                    preferred_element_type=jnp.float32, precision=lax.Precision.HIGHEST)

    @pl.when(j == nsteps - 1)
    def _():
        acc = acc_ref[...]
        c = acc[:, :N_FEAT] / jnp.maximum(acc[:, N_FEAT:N_FEAT + 1], 1.0)
        cc = jnp.maximum(
            jnp.dot(c, fcw_ref[...], preferred_element_type=jnp.float32,
                    precision=lax.Precision.DEFAULT)
            + fcb_ref[...], 0.0)
        hb = jnp.maximum(
            jnp.dot(cc, w1b_ref[...], preferred_element_type=jnp.float32,
                    precision=lax.Precision.DEFAULT)
            + b1b_ref[...], 0.0)
        obg_ref[...] = jnp.dot(
            hb, w2b_ref[...], preferred_element_type=jnp.float32,
            precision=lax.Precision.DEFAULT) + b2b_ref[...]
        he = jnp.maximum(
            jnp.dot(cc, w1e_ref[...], preferred_element_type=jnp.float32,
                    precision=lax.Precision.DEFAULT)
            + b1e_ref[...], 0.0)
        oeh_ref[...] = jnp.dot(
            he, w2e_ref[...], preferred_element_type=jnp.float32,
            precision=lax.Precision.DEFAULT) + b2e_ref[...]


def _pool_heads(h, agg, bc, fcw, fcb, hb, he):
    n = N_NODES // P_BLK
    full = lambda shape: pl.BlockSpec(shape, lambda j: tuple(0 for _ in shape))
    return pl.pallas_call(
        _pool_body,
        grid=(n,),
        in_specs=[
            pl.BlockSpec((P_BLK, N_FEAT), lambda j: (j, 0)),
            pl.BlockSpec((2, P_BLK, HALF), lambda j: (0, j, 0)),
            pl.BlockSpec((P_BLK, 1), lambda j: (j, 0)),
            full((N_FEAT, 2 * N_FEAT)), full((1, 2 * N_FEAT)),
            full((2 * N_FEAT, N_FEAT)), full((1, N_FEAT)),
            full((N_FEAT, 1)), full((1, 1)),
            full((2 * N_FEAT, N_FEAT)), full((1, N_FEAT)),
            full((N_FEAT, 1)), full((1, 1)),
        ],
        out_specs=[
            pl.BlockSpec((N_GRAPHS, 1), lambda j: (0, 0)),
            pl.BlockSpec((N_GRAPHS, 1), lambda j: (0, 0)),
        ],
        out_shape=[
            jax.ShapeDtypeStruct((N_GRAPHS, 1), jnp.float32),
            jax.ShapeDtypeStruct((N_GRAPHS, 1), jnp.float32),
        ],
        scratch_shapes=[pltpu.VMEM((N_GRAPHS, N_FEAT + 8), jnp.float32)],
    )(h, agg, bc, fcw, fcb,
      hb['W1'], hb['b1'].reshape(1, -1), hb['W2'], hb['b2'].reshape(1, 1),
      he['W1'], he['b1'].reshape(1, -1), he['W2'], he['b2'].reshape(1, 1))


def _sc_block_body(blk, srcq_hbm, dst_hbm, msg_hbm, gate_hbm, agg_hbm,
                   sidx, didx, grow, gatev, accum,
                   sem_i, sem_a0, sem_a1, sem_g0, sem_g1):
    c = lax.axis_index("c")
    s = lax.axis_index("s")

    zeros16 = jnp.zeros((16,), jnp.float32)

    def _zb(e, carry):
        grow[e, pl.ds(0, 16)] = zeros16
        grow[e, pl.ds(16, 16)] = zeros16
        return carry
    lax.fori_loop(0, 2 * ROW_W, _zb, 0, unroll=4)
    zsz = 2 * ROW_W
    for t in range(ACC_STRIPE // zsz):
        pltpu.sync_copy(grow.at[pl.ds(0, zsz)],
                        accum.at[pl.ds(s * ACC_STRIPE + t * zsz, zsz)])
    rem = ACC_STRIPE % zsz
    if rem:
        pltpu.sync_copy(
            grow.at[pl.ds(0, rem)],
            accum.at[pl.ds(s * ACC_STRIPE + ACC_STRIPE - rem, rem)])
    plsc.subcore_barrier()

    sem_as = (sem_a0, sem_a1)
    sem_gs = (sem_g0, sem_g1)
    NB = ROWS_PER_SUB // 8
    gate_block = blk * 2 * (E_PAD // 4) + 0

    def _idx_copies(b, p):
        row = s * ROWS_PER_SUB + b * 8
        sc = pltpu.make_async_copy(
            srcq_hbm.at[pl.ds((c * N_ROWS + row) * ROW_W, 8 * ROW_W)],
            sidx.at[pl.ds(p * 8 * ROW_W, 8 * ROW_W)], sem_i)
        dc = pltpu.make_async_copy(dst_hbm.at[pl.ds(row, 8)], didx.at[p],
                                   sem_i)
        return sc, dc

    sc0, dc0 = _idx_copies(0, 0)
    sc0.start(); sc0.wait()
    dc0.start(); dc0.wait()

    def _block(b, carry):
        p = lax.rem(b, 2)
        @pl.when(b + 1 < NB)
        def _():
            sc, dc = _idx_copies(b + 1, 1 - p)
            sc.start()
            dc.start()

        rb = s * ROWS_PER_SUB + b * 8

        def _fire(k):
            slot = k % 2
            gbase = (gate_block + c * (E_PAD // 4) + (rb + k) * (ROW_W // 4)
                     ) * 128
            gd = pltpu.async_copy(
                gate_hbm.at[pl.ds(gbase, (ROW_W // 4) * 128)],
                gatev.at[pl.ds(slot * 4096, 4096)], sem_gs[slot])
            ad = pltpu.async_copy(
                msg_hbm.at[sidx.at[pl.ds(p * 8 * ROW_W + k * ROW_W, ROW_W)]],
                grow.at[pl.ds(slot * ROW_W, ROW_W)], sem_as[slot])
            return gd, ad

        inflight = {0: _fire(0), 1: _fire(1)}
        for k in range(8):
            slot = k % 2
            gd, ad = inflight.pop(k)
            gd.wait()
            ad.wait()

            def _mul(t, carry):
                for u in range(4):
                    for q in range(HALF // 16):
                        slm = pl.ds(q * 16, 16)
                        slg = pl.ds(slot * 4096 + 128 * t + u * HALF + q * 16,
                                    16)
                        e = slot * ROW_W + 4 * t + u
                        grow[e, slm] = grow[e, slm] * gatev[slg]
                return carry
            lax.fori_loop(0, ROW_W // 4, _mul, 0, unroll=4)
            pltpu.sync_copy(grow.at[pl.ds(slot * ROW_W, ROW_W)],
                            accum.at[didx.at[p, k]], add=True)
            if k + 2 < 8:
                inflight[k + 2] = _fire(k + 2)

        @pl.when(b + 1 < NB)
        def _():
            sc, dc = _idx_copies(b + 1, 1 - p)
            sc.wait()
            dc.wait()
        return carry

    lax.fori_loop(0, NB, _block, 0)

    plsc.subcore_barrier()
    pltpu.sync_copy(
        accum.at[pl.ds(s * ACC_STRIPE, ACC_STRIPE)],
        agg_hbm.at[pl.ds(c * ACC_ROWS + s * ACC_STRIPE, ACC_STRIPE)])


def _sc_block(blk):
    mesh = plsc.VectorSubcoreMesh(core_axis_name="c", subcore_axis_name="s")
    return pl.kernel(
        functools.partial(_sc_block_body, blk),
        out_type=jax.ShapeDtypeStruct((2 * ACC_ROWS, HALF), jnp.float32),
        mesh=mesh,
        scratch_types=[
            pltpu.VMEM((2 * 8 * ROW_W,), jnp.int32),
            pltpu.VMEM((2, 8, ROW_W), jnp.int32),
            pltpu.VMEM((2 * ROW_W, HALF), jnp.float32),
            pltpu.VMEM((2 * 4096,), jnp.float32),
            pltpu.VMEM_SHARED((ACC_ROWS, HALF), jnp.float32),
            pltpu.SemaphoreType.DMA,
            pltpu.SemaphoreType.DMA,
            pltpu.SemaphoreType.DMA,
            pltpu.SemaphoreType.DMA,
            pltpu.SemaphoreType.DMA,
        ],
        compiler_params=pltpu.CompilerParams(use_tc_tiling_on_sc=False),
        name=f"mp_block{blk}",
    )


def kernel(x, edge_index, edge_attr, batch, emb, params):
    npad = E_PAD - N_EDGES
    src1 = jnp.concatenate(
        [edge_index[0].astype(jnp.int32), jnp.zeros((npad,), jnp.int32)])
    srcq = jnp.concatenate([src1, src1 + N_NODES])
    dst2d = jnp.concatenate(
        [edge_index[1].astype(jnp.int32),
         jnp.full((npad,), N_NODES, jnp.int32)]
    ).reshape(N_ROWS, ROW_W)
    dcol = jnp.concatenate(
        [edge_attr.astype(jnp.float32), jnp.zeros((npad,), jnp.float32)]
    ).reshape(E_PAD // 4, 4)
    xcol = x.astype(jnp.int32).reshape(N_NODES, 1)
    bcol = batch.astype(jnp.int32).reshape(N_NODES, 1)

    embp = jnp.zeros((128, N_FEAT), jnp.float32).at[:emb.shape[0]].set(emb)
    wep = jnp.stack([
        jnp.zeros((128, N_FEAT), jnp.float32)
        .at[:N_RBF].set(params['block' + str(i)]['We'])
        for i in range(N_CONV)])
    bes = jnp.stack([params['block' + str(i)]['be'] for i in range(N_CONV)])

    gates = _gates(dcol, wep, bes).reshape(-1)

    h, msg = _embed(xcol, embp, params['block0']['Wn'],
                    params['block0']['bn'].reshape(1, N_FEAT))

    agg = None
    for i in range(N_CONV):
        aggf = _sc_block(i)(srcq, dst2d, msg.reshape(2 * N_NODES, HALF),
                            gates)
        agg = aggf.reshape(2, ACC_ROWS, HALF)
        if i + 1 < N_CONV:
            p = params['block' + str(i + 1)]
            h, msg = _update(h, agg, p['Wn'], p['bn'].reshape(1, N_FEAT))

    out_bg, out_eh = _pool_heads(h, agg, bcol, params['fc_W'],
                                 params['fc_b'].reshape(1, -1),
                                 params['head_bg'], params['head_eh'])
    return (out_bg, out_eh)
